# baseline (speedup 1.0000x reference)
import jax
import jax.numpy as jnp
from jax.experimental import pallas as pl

N = 10000


def _bn(h, g, b):
    m = h.mean(axis=0)
    v = h.var(axis=0)
    return (h - m) / jnp.sqrt(v + 1e-5) * g + b


def _sage(x, ei, Wl, bl, Wr):
    src = ei[0]
    dst = ei[1]
    msgs = jnp.take(x, src, axis=0)
    summed = jax.ops.segment_sum(msgs, dst, num_segments=N)
    cnt = jax.ops.segment_sum(jnp.ones((ei.shape[1],), jnp.float32), dst, num_segments=N)
    aggr = summed / jnp.maximum(cnt, 1.0)[:, None]
    return aggr @ Wl.T + bl + x @ Wr.T


def _copy_kernel(x_ref, o_ref):
    o_ref[...] = x_ref[...]


def kernel(x_c1, x_be_edge_index, x_ge_edge_index, W1, b1, g1, be1, W2, b2, g2, be2, Wl_be, bl_be, Wr_be, Wl_ge, bl_ge, Wr_ge, Wm1, bm1, Wm2, bm2, Ws, bs, Wd, bd, disp, Wb, bb, Wg, bg):
    h = jax.nn.relu(_bn(x_c1 @ W1.T + b1, g1, be1))
    z_mix = jax.nn.relu(_bn(h @ W2.T + b2, g2, be2))
    z_mix = pl.pallas_call(
        _copy_kernel,
        out_shape=jax.ShapeDtypeStruct(z_mix.shape, z_mix.dtype),
    )(z_mix)
    z_be = _sage(z_mix, x_be_edge_index, Wl_be, bl_be, Wr_be)
    z_ge = _sage(z_mix, x_ge_edge_index, Wl_ge, bl_ge, Wr_ge)
    cat1 = jnp.concatenate((z_mix, z_be, z_ge), axis=1)
    z_un = jax.nn.relu(cat1 @ Wm1.T + bm1) @ Wm2.T + bm2
    h2 = jnp.concatenate((z_be, z_ge, z_un), axis=1)
    px_scale = jnp.exp(h2 @ Ws.T + bs)
    px_dropout = h2 @ Wd.T + bd
    px_rate = jnp.exp(disp)
    batch_pred = z_be @ Wb.T + bb
    group_pred = z_ge @ Wg.T + bg
    return (z_mix, z_be, z_ge, z_un, px_scale, px_rate, px_dropout, batch_pred, group_pred)


# speedup vs baseline: 5.5758x; 5.5758x over previous
import functools

import jax
import jax.numpy as jnp
from jax import lax
from jax.experimental import pallas as pl
from jax.experimental.pallas import tpu as pltpu
from jax.experimental.pallas import tpu_sc as plsc

_N = 10000
_E = 320000
_D_IN = 512
_H1 = 256
_LD = 128
_EPS = 1e-5

_R = 1000
_G = _N // _R

_NT = 16
_CH = 128
_NCH = 156
_EPT = _NCH * _CH
_TAIL = _E - _NT * _EPT
_NP = 10240
_RPT = _NP // _NT


def _enc1_body(x_ref, w1t_ref, b1_ref, h_ref, s_ref, ss_ref):
    i = pl.program_id(0)
    h = jnp.dot(x_ref[...], w1t_ref[...], preferred_element_type=jnp.float32)
    h = h + b1_ref[...]
    h_ref[...] = h
    s = jnp.sum(h, axis=0, keepdims=True)
    ss = jnp.sum(h * h, axis=0, keepdims=True)

    @pl.when(i == 0)
    def _():
        s_ref[...] = s
        ss_ref[...] = ss

    @pl.when(i > 0)
    def _():
        s_ref[...] += s
        ss_ref[...] += ss


def _enc2_body(h_ref, s1_ref, ss1_ref, g1_ref, be1_ref, w2t_ref, b2_ref,
               z0_ref, s2_ref, ss2_ref):
    i = pl.program_id(0)
    m = s1_ref[...] / _N
    v = ss1_ref[...] / _N - m * m
    hn = (h_ref[...] - m) * lax.rsqrt(v + _EPS) * g1_ref[...] + be1_ref[...]
    hn = jnp.maximum(hn, 0.0)
    z0 = jnp.dot(hn, w2t_ref[...], preferred_element_type=jnp.float32)
    z0 = z0 + b2_ref[...]
    z0_ref[...] = z0
    s = jnp.sum(z0, axis=0, keepdims=True)
    ss = jnp.sum(z0 * z0, axis=0, keepdims=True)

    @pl.when(i == 0)
    def _():
        s2_ref[...] = s
        ss2_ref[...] = ss

    @pl.when(i > 0)
    def _():
        s2_ref[...] += s
        ss2_ref[...] += ss


def _enc3_body(z0_ref, s2_ref, ss2_ref, g2_ref, be2_ref, wrbet_ref, wrget_ref,
               zmix_ref, rbe_ref, rge_ref):
    m = s2_ref[...] / _N
    v = ss2_ref[...] / _N - m * m
    z = (z0_ref[...] - m) * lax.rsqrt(v + _EPS) * g2_ref[...] + be2_ref[...]
    z = jnp.maximum(z, 0.0)
    zmix_ref[...] = z
    rbe_ref[...] = jnp.dot(z, wrbet_ref[...], preferred_element_type=jnp.float32)
    rge_ref[...] = jnp.dot(z, wrget_ref[...], preferred_element_type=jnp.float32)


def _sage_sc_body(zmix, be_src, be_dst, ge_src, ge_dst, sum_out, cnt_out,
                  idx_src, idx_dst, rows, cnt1d, sem, ssum):
    c = lax.axis_index("c")
    s = lax.axis_index("s")

    def _zrow(k, carry):
        i = k // 8
        j = k - i * 8
        rows[i, pl.ds(j * 16, 16)] = jnp.zeros((16,), jnp.float32)
        return carry
    lax.fori_loop(0, _CH * 8, _zrow, 0)

    def _zc(i, carry):
        cnt1d[pl.ds(i * 16, 16)] = jnp.zeros((16,), jnp.float32)
        return carry
    lax.fori_loop(0, _NP // 16, _zc, 0)

    for i in range(_RPT // _CH):
        pltpu.sync_copy(rows, ssum.at[pl.ds(s * _RPT + i * _CH, _CH)])
    plsc.subcore_barrier()

    ones16 = jnp.ones((16,), jnp.float32)

    def _do_chunk(src_ref, dst_ref, b):
        pltpu.sync_copy(src_ref.at[pl.ds(b, _CH)], idx_src)
        pltpu.sync_copy(dst_ref.at[pl.ds(b, _CH)], idx_dst)
        pltpu.async_copy(zmix.at[idx_src], rows, sem).wait()
        pltpu.sync_copy(rows, ssum.at[idx_dst], add=True)

        def _cstep(j, carry):
            d = idx_dst[pl.ds(j * 16, 16)]
            plsc.addupdate_scatter(cnt1d, [d], ones16)
            return carry
        lax.fori_loop(0, _CH // 16, _cstep, 0)

    def _run_set(src_ref, dst_ref):
        ebase = s * _EPT

        def _chunk(i, carry):
            _do_chunk(src_ref, dst_ref, pl.multiple_of(ebase + i * _CH, _CH))
            return carry
        lax.fori_loop(0, _NCH, _chunk, 0)

        @pl.when(s < _TAIL // _CH)
        def _():
            _do_chunk(src_ref, dst_ref,
                      pl.multiple_of(_NT * _EPT + s * _CH, _CH))

    @pl.when(c == 0)
    def _():
        _run_set(be_src, be_dst)

    @pl.when(c == 1)
    def _():
        _run_set(ge_src, ge_dst)

    plsc.subcore_barrier()

    obase = pl.multiple_of(c * _NP + s * _RPT, 8)
    pltpu.sync_copy(ssum.at[pl.ds(s * _RPT, _RPT)], sum_out.at[pl.ds(obase, _RPT)])
    w = c * _NT + s
    pltpu.sync_copy(cnt1d, cnt_out.at[pl.ds(w * _NP, _NP)])


def _dec_body(zmix_ref, rbe_ref, rge_ref, sbe_ref, cbe_ref, sge_ref, cge_ref,
              wlbet_ref, blbe_ref, wlget_ref, blge_ref,
              wm1t_ref, bm1_ref, wm2t_ref, bm2_ref,
              wst_ref, bs_ref, wdt_ref, bd_ref, disp_ref,
              wbt_ref, bb_ref, wgt_ref, bg_ref,
              zbe_ref, zge_ref, zun_ref, pxs_ref, pxr_ref, pxd_ref,
              bp_ref, gp_ref):
    cbe = jnp.maximum(jnp.sum(cbe_ref[...], axis=1, keepdims=True), 1.0)
    cge = jnp.maximum(jnp.sum(cge_ref[...], axis=1, keepdims=True), 1.0)
    abe = sbe_ref[...] / cbe
    age = sge_ref[...] / cge
    zbe = jnp.dot(abe, wlbet_ref[...], preferred_element_type=jnp.float32)
    zbe = zbe + blbe_ref[...] + rbe_ref[...]
    zge = jnp.dot(age, wlget_ref[...], preferred_element_type=jnp.float32)
    zge = zge + blge_ref[...] + rge_ref[...]
    zbe_ref[...] = zbe
    zge_ref[...] = zge

    zmix = zmix_ref[...]
    u = (jnp.dot(zmix, wm1t_ref[0:_LD], preferred_element_type=jnp.float32)
         + jnp.dot(zbe, wm1t_ref[_LD:2 * _LD], preferred_element_type=jnp.float32)
         + jnp.dot(zge, wm1t_ref[2 * _LD:3 * _LD], preferred_element_type=jnp.float32)
         + bm1_ref[...])
    u = jnp.maximum(u, 0.0)
    zun = jnp.dot(u, wm2t_ref[...], preferred_element_type=jnp.float32) + bm2_ref[...]
    zun_ref[...] = zun

    ls = (jnp.dot(zbe, wst_ref[0:_LD], preferred_element_type=jnp.float32)
          + jnp.dot(zge, wst_ref[_LD:2 * _LD], preferred_element_type=jnp.float32)
          + jnp.dot(zun, wst_ref[2 * _LD:3 * _LD], preferred_element_type=jnp.float32)
          + bs_ref[...])
    pxs_ref[...] = jnp.exp(ls)
    pxd_ref[...] = (jnp.dot(zbe, wdt_ref[0:_LD], preferred_element_type=jnp.float32)
                    + jnp.dot(zge, wdt_ref[_LD:2 * _LD], preferred_element_type=jnp.float32)
                    + jnp.dot(zun, wdt_ref[2 * _LD:3 * _LD], preferred_element_type=jnp.float32)
                    + bd_ref[...])
    pxr_ref[...] = jnp.exp(disp_ref[...])
    bp_ref[...] = jnp.dot(zbe, wbt_ref[...], preferred_element_type=jnp.float32) + bb_ref[...]
    gp_ref[...] = jnp.dot(zge, wgt_ref[...], preferred_element_type=jnp.float32) + bg_ref[...]


def _sc_aggregate(z_mix, be_ei, ge_ei):
    f32 = jnp.float32
    sc = functools.partial(
        pl.kernel,
        out_type=[jax.ShapeDtypeStruct((2 * _NP, _LD), f32),
                  jax.ShapeDtypeStruct((2 * _NT * _NP,), f32)],
        mesh=plsc.VectorSubcoreMesh(core_axis_name="c", subcore_axis_name="s",
                                    num_cores=2, num_subcores=_NT),
        compiler_params=pltpu.CompilerParams(needs_layout_passes=False),
        scratch_types=[
            pltpu.VMEM((_CH,), jnp.int32),
            pltpu.VMEM((_CH,), jnp.int32),
            pltpu.VMEM((_CH, _LD), f32),
            pltpu.VMEM((_NP,), f32),
            pltpu.SemaphoreType.DMA,
            pltpu.VMEM_SHARED((_NP, _LD), f32),
        ],
    )(_sage_sc_body)
    return sc(z_mix, be_ei[0], be_ei[1], ge_ei[0], ge_ei[1])


def _row_spec(width):
    return pl.BlockSpec((_R, width), lambda i: (i, 0))


def _full_spec(shape):
    nd = len(shape)
    return pl.BlockSpec(shape, lambda i: (0,) * nd)


def kernel(x_c1, x_be_edge_index, x_ge_edge_index, W1, b1, g1, be1, W2, b2, g2, be2, Wl_be, bl_be, Wr_be, Wl_ge, bl_ge, Wr_ge, Wm1, bm1, Wm2, bm2, Ws, bs, Wd, bd, disp, Wb, bb, Wg, bg):
    f32 = jnp.float32
    row = lambda v: v.reshape(1, -1)

    h, s1, ss1 = pl.pallas_call(
        _enc1_body,
        grid=(_G,),
        in_specs=[_row_spec(_D_IN), _full_spec((_D_IN, _H1)), _full_spec((1, _H1))],
        out_specs=[_row_spec(_H1), _full_spec((1, _H1)), _full_spec((1, _H1))],
        out_shape=[jax.ShapeDtypeStruct((_N, _H1), f32),
                   jax.ShapeDtypeStruct((1, _H1), f32),
                   jax.ShapeDtypeStruct((1, _H1), f32)],
    )(x_c1, W1.T, row(b1))

    z0, s2, ss2 = pl.pallas_call(
        _enc2_body,
        grid=(_G,),
        in_specs=[_row_spec(_H1), _full_spec((1, _H1)), _full_spec((1, _H1)),
                  _full_spec((1, _H1)), _full_spec((1, _H1)),
                  _full_spec((_H1, _LD)), _full_spec((1, _LD))],
        out_specs=[_row_spec(_LD), _full_spec((1, _LD)), _full_spec((1, _LD))],
        out_shape=[jax.ShapeDtypeStruct((_N, _LD), f32),
                   jax.ShapeDtypeStruct((1, _LD), f32),
                   jax.ShapeDtypeStruct((1, _LD), f32)],
    )(h, s1, ss1, row(g1), row(be1), W2.T, row(b2))

    z_mix, r_be, r_ge = pl.pallas_call(
        _enc3_body,
        grid=(_G,),
        in_specs=[_row_spec(_LD), _full_spec((1, _LD)), _full_spec((1, _LD)),
                  _full_spec((1, _LD)), _full_spec((1, _LD)),
                  _full_spec((_LD, _LD)), _full_spec((_LD, _LD))],
        out_specs=[_row_spec(_LD), _row_spec(_LD), _row_spec(_LD)],
        out_shape=[jax.ShapeDtypeStruct((_N, _LD), f32),
                   jax.ShapeDtypeStruct((_N, _LD), f32),
                   jax.ShapeDtypeStruct((_N, _LD), f32)],
    )(z0, s2, ss2, row(g2), row(be2), Wr_be.T, Wr_ge.T)

    sums, cntp = _sc_aggregate(z_mix, x_be_edge_index, x_ge_edge_index)
    sum_be, sum_ge = sums[:_N], sums[_NP:_NP + _N]
    cntp = cntp.reshape(2, _NT, _NP)
    cnt_be = cntp[0].T[:_N]
    cnt_ge = cntp[1].T[:_N]

    outs = pl.pallas_call(
        _dec_body,
        grid=(_G,),
        in_specs=[_row_spec(_LD), _row_spec(_LD), _row_spec(_LD),
                  _row_spec(_LD), pl.BlockSpec((_R, 16), lambda i: (i, 0)),
                  _row_spec(_LD), pl.BlockSpec((_R, 16), lambda i: (i, 0)),
                  _full_spec((_LD, _LD)), _full_spec((1, _LD)),
                  _full_spec((_LD, _LD)), _full_spec((1, _LD)),
                  _full_spec((3 * _LD, _LD)), _full_spec((1, _LD)),
                  _full_spec((_LD, _LD)), _full_spec((1, _LD)),
                  _full_spec((3 * _LD, _D_IN)), _full_spec((1, _D_IN)),
                  _full_spec((3 * _LD, _D_IN)), _full_spec((1, _D_IN)),
                  _full_spec((1, _D_IN)),
                  _full_spec((_LD, 8)), _full_spec((1, 8)),
                  _full_spec((_LD, 16)), _full_spec((1, 16))],
        out_specs=[_row_spec(_LD), _row_spec(_LD), _row_spec(_LD),
                   _row_spec(_D_IN), _full_spec((1, _D_IN)), _row_spec(_D_IN),
                   pl.BlockSpec((_R, 8), lambda i: (i, 0)),
                   pl.BlockSpec((_R, 16), lambda i: (i, 0))],
        out_shape=[jax.ShapeDtypeStruct((_N, _LD), f32),
                   jax.ShapeDtypeStruct((_N, _LD), f32),
                   jax.ShapeDtypeStruct((_N, _LD), f32),
                   jax.ShapeDtypeStruct((_N, _D_IN), f32),
                   jax.ShapeDtypeStruct((1, _D_IN), f32),
                   jax.ShapeDtypeStruct((_N, _D_IN), f32),
                   jax.ShapeDtypeStruct((_N, 8), f32),
                   jax.ShapeDtypeStruct((_N, 16), f32)],
    )(z_mix, r_be, r_ge, sum_be, cnt_be, sum_ge, cnt_ge,
      Wl_be.T, row(bl_be), Wl_ge.T, row(bl_ge),
      Wm1.T, row(bm1), Wm2.T, row(bm2),
      Ws.T, row(bs), Wd.T, row(bd), row(disp),
      Wb.T, row(bb), Wg.T, row(bg))
    z_be, z_ge, z_un, px_scale, px_rate2d, px_dropout, batch_pred, group_pred = outs
    return (z_mix, z_be, z_ge, z_un, px_scale, px_rate2d.reshape(_D_IN),
            px_dropout, batch_pred, group_pred)
